# baseline (speedup 1.0000x reference)
import functools

import jax
import jax.numpy as jnp
from jax import lax
from jax.experimental import pallas as pl
from jax.experimental.pallas import tpu as pltpu
from jax.experimental.pallas import tpu_sc as plsc

MAXZ1 = 102
NCFG = 20
D = 64
B = 4096
S = 200
N = B * S
TROWS = 208
EMB_OFF = MAXZ1 * D
NC = 2
NS = 16
NW = NC * NS
BW = B // NW
SCH = 4
NCHUNK = S // SCH


def _build_table(embedding_weight, electron_config, config_w, config_b):
    emb_p = (jnp.zeros((TROWS, D), jnp.float32)
             .at[:MAXZ1].set(embedding_weight)
             .at[MAXZ1:2 * MAXZ1].set(embedding_weight))
    ec_p = jnp.zeros((TROWS, 128), jnp.float32).at[:MAXZ1, :NCFG].set(electron_config)
    wt_p = jnp.zeros((128, D), jnp.float32).at[:NCFG].set(config_w.T)
    b_p = jnp.broadcast_to(config_b[None, :], (8, D))
    bmask = (lax.broadcasted_iota(jnp.int32, (TROWS, 1), 0) < MAXZ1).astype(jnp.float32)
    def body(emb_ref, ec_ref, wt_ref, b_ref, m_ref, out_ref):
        elec = jnp.dot(ec_ref[...], wt_ref[...], preferred_element_type=jnp.float32)
        out_ref[...] = emb_ref[...] + elec + b_ref[0:1, :] * m_ref[...]
    return pl.pallas_call(
        body,
        out_shape=jax.ShapeDtypeStruct((TROWS, D), jnp.float32),
    )(emb_p, ec_p, wt_p, b_p, bmask)


def _gather_body(tab_hbm, idx_hbm, out_hbm, tab_v, idx_v, out_v, sem):
    wid = lax.axis_index("s") * NC + lax.axis_index("c")
    b0 = wid * BW
    pltpu.sync_copy(tab_hbm, tab_v)
    pltpu.sync_copy(idx_hbm.at[pl.ds(b0 * S, BW * S)], idx_v)
    lanes = lax.broadcasted_iota(jnp.int32, (16,), 0)

    def chunk(c, carry):
        for s_loc in range(SCH):
            s = c * SCH + s_loc
            off = jnp.where(s == 0, EMB_OFF, 0)
            def bgroup(bg, carry2):
                bvec = bg * 16 + lanes
                z16 = plsc.load_gather(idx_v, [bvec * S + s])
                base = z16 * D + off
                for d in range(D):
                    vals = plsc.load_gather(tab_v, [base + d])
                    out_v[s_loc, d, pl.ds(bg * 16, 16)] = vals
                return carry2
            lax.fori_loop(0, BW // 16, bgroup, 0)
        pltpu.async_copy(
            out_v,
            out_hbm.at[pl.ds(c * SCH, SCH), :, pl.ds(b0, BW)],
            sem,
        ).wait()
        return carry

    lax.fori_loop(0, NCHUNK, chunk, 0)


@functools.partial(
    pl.kernel,
    mesh=plsc.VectorSubcoreMesh(core_axis_name="c", subcore_axis_name="s"),
    compiler_params=pltpu.CompilerParams(needs_layout_passes=False),
    out_type=jax.ShapeDtypeStruct((S, D, B), jnp.float32),
    scratch_types=[
        pltpu.VMEM((TROWS * D,), jnp.float32),
        pltpu.VMEM((BW * S,), jnp.int32),
        pltpu.VMEM((SCH, D, BW), jnp.float32),
        pltpu.SemaphoreType.DMA,
    ],
)
def _sc_gather_t(tab_hbm, idx_hbm, out_hbm, tab_v, idx_v, out_v, sem):
    _gather_body(tab_hbm, idx_hbm, out_hbm, tab_v, idx_v, out_v, sem)


def kernel(atomic_numbers, embedding_weight, electron_config, config_w, config_b):
    table = _build_table(embedding_weight, electron_config, config_w, config_b)
    out_t = _sc_gather_t(table.reshape(TROWS * D), atomic_numbers.reshape(N))
    return jnp.transpose(out_t, (2, 0, 1))


# speedup vs baseline: 1.4636x; 1.4636x over previous
import functools

import jax
import jax.numpy as jnp
from jax import lax
from jax.experimental import pallas as pl
from jax.experimental.pallas import tpu as pltpu
from jax.experimental.pallas import tpu_sc as plsc

MAXZ1 = 102
NCFG = 20
D = 64
B = 4096
S = 200
N = B * S
TROWS = 208
EMB_OFF = MAXZ1 * D
NC = 2
NS = 16
NW = NC * NS
BW = B // NW
SCH = 4
NCHUNK = S // SCH


def _build_table(embedding_weight, electron_config, config_w, config_b):
    emb_p = (jnp.zeros((TROWS, D), jnp.float32)
             .at[:MAXZ1].set(embedding_weight)
             .at[MAXZ1:2 * MAXZ1].set(embedding_weight))
    ec_p = jnp.zeros((TROWS, 128), jnp.float32).at[:MAXZ1, :NCFG].set(electron_config)
    wt_p = jnp.zeros((128, D), jnp.float32).at[:NCFG].set(config_w.T)
    b_p = jnp.broadcast_to(config_b[None, :], (8, D))
    bmask = (lax.broadcasted_iota(jnp.int32, (TROWS, 1), 0) < MAXZ1).astype(jnp.float32)
    def body(emb_ref, ec_ref, wt_ref, b_ref, m_ref, out_ref):
        elec = jnp.dot(ec_ref[...], wt_ref[...], preferred_element_type=jnp.float32)
        out_ref[...] = emb_ref[...] + elec + b_ref[0:1, :] * m_ref[...]
    return pl.pallas_call(
        body,
        out_shape=jax.ShapeDtypeStruct((TROWS, D), jnp.float32),
    )(emb_p, ec_p, wt_p, b_p, bmask)


def _gather_body(tab_hbm, idx_hbm, out_hbm, tab_v, idx_v, out_v, sem):
    wid = lax.axis_index("s") * NC + lax.axis_index("c")
    b0 = wid * BW
    pltpu.sync_copy(tab_hbm, tab_v)
    pltpu.sync_copy(idx_hbm.at[pl.ds(b0 * S, BW * S)], idx_v)
    lanes = lax.broadcasted_iota(jnp.int32, (16,), 0)

    def chunk(c, carry):
        for s_loc in range(SCH):
            s = c * SCH + s_loc
            off = jnp.where(s == 0, EMB_OFF, 0)
            @plsc.parallel_loop(0, BW // 16, unroll=2)
            def bgroup(bg):
                bvec = bg * 16 + lanes
                z16 = plsc.load_gather(idx_v, [bvec * S + s])
                base = z16 * D + off
                for d in range(D):
                    vals = plsc.load_gather(tab_v, [base + d])
                    out_v[s_loc, d, pl.ds(bg * 16, 16)] = vals
        pltpu.async_copy(
            out_v,
            out_hbm.at[pl.ds(c * SCH, SCH), :, pl.ds(b0, BW)],
            sem,
        ).wait()
        return carry

    lax.fori_loop(0, NCHUNK, chunk, 0)


@functools.partial(
    pl.kernel,
    mesh=plsc.VectorSubcoreMesh(core_axis_name="c", subcore_axis_name="s"),
    compiler_params=pltpu.CompilerParams(needs_layout_passes=False,
                                         disable_bounds_checks=True),
    out_type=jax.ShapeDtypeStruct((S, D, B), jnp.float32),
    scratch_types=[
        pltpu.VMEM((TROWS * D,), jnp.float32),
        pltpu.VMEM((BW * S,), jnp.int32),
        pltpu.VMEM((SCH, D, BW), jnp.float32),
        pltpu.SemaphoreType.DMA,
    ],
)
def _sc_gather_t(tab_hbm, idx_hbm, out_hbm, tab_v, idx_v, out_v, sem):
    _gather_body(tab_hbm, idx_hbm, out_hbm, tab_v, idx_v, out_v, sem)


def kernel(atomic_numbers, embedding_weight, electron_config, config_w, config_b):
    table = _build_table(embedding_weight, electron_config, config_w, config_b)
    out_t = _sc_gather_t(table.reshape(TROWS * D), atomic_numbers.reshape(N))
    return jnp.transpose(out_t, (2, 0, 1))


# speedup vs baseline: 2.9425x; 2.0105x over previous
import functools

import jax
import jax.numpy as jnp
from jax import lax
from jax.experimental import pallas as pl
from jax.experimental.pallas import tpu as pltpu
from jax.experimental.pallas import tpu_sc as plsc

MAXZ1 = 102
NCFG = 20
D = 64
B = 4096
S = 200
N = B * S
TROWS = 208
TSTR = 65
EMB_OFF = MAXZ1 * TSTR
NC = 2
NS = 16
NW = NC * NS
BW = B // NW
SCH = 4
NCHUNK = S // SCH


def _build_table(embedding_weight, electron_config, config_w, config_b):
    emb_p = (jnp.zeros((TROWS, D), jnp.float32)
             .at[:MAXZ1].set(embedding_weight)
             .at[MAXZ1:2 * MAXZ1].set(embedding_weight))
    ec_p = jnp.zeros((TROWS, 128), jnp.float32).at[:MAXZ1, :NCFG].set(electron_config)
    wt_p = jnp.zeros((128, D), jnp.float32).at[:NCFG].set(config_w.T)
    b_p = jnp.broadcast_to(config_b[None, :], (8, D))
    bmask = (lax.broadcasted_iota(jnp.int32, (TROWS, 1), 0) < MAXZ1).astype(jnp.float32)
    def body(emb_ref, ec_ref, wt_ref, b_ref, m_ref, out_ref):
        elec = jnp.dot(ec_ref[...], wt_ref[...], preferred_element_type=jnp.float32)
        out_ref[:, 0:D] = emb_ref[...] + elec + b_ref[0:1, :] * m_ref[...]
        out_ref[:, D:TSTR] = jnp.zeros((TROWS, TSTR - D), jnp.float32)
    return pl.pallas_call(
        body,
        out_shape=jax.ShapeDtypeStruct((TROWS, TSTR), jnp.float32),
    )(emb_p, ec_p, wt_p, b_p, bmask)


def _gather_body(tab_hbm, idx_hbm, out_hbm, tab_v, idx_v, out_v, sem):
    wid = lax.axis_index("s") * NC + lax.axis_index("c")
    b0 = wid * BW
    pltpu.sync_copy(tab_hbm, tab_v)
    pltpu.sync_copy(idx_hbm.at[pl.ds(b0 * S, BW * S)], idx_v)
    lanes = lax.broadcasted_iota(jnp.int32, (16,), 0)

    def chunk(c, carry):
        for s_loc in range(SCH):
            s = c * SCH + s_loc
            off = jnp.where(s == 0, EMB_OFF, 0)
            @plsc.parallel_loop(0, BW // 16, unroll=2)
            def bgroup(bg):
                bvec = bg * 16 + lanes
                z16 = plsc.load_gather(idx_v, [bvec * S + s])
                base = z16 * TSTR + off
                for d in range(D):
                    vals = plsc.load_gather(tab_v, [base + d])
                    out_v[s_loc, d, pl.ds(bg * 16, 16)] = vals
        pltpu.async_copy(
            out_v,
            out_hbm.at[pl.ds(c * SCH, SCH), :, pl.ds(b0, BW)],
            sem,
        ).wait()
        return carry

    lax.fori_loop(0, NCHUNK, chunk, 0)


@functools.partial(
    pl.kernel,
    mesh=plsc.VectorSubcoreMesh(core_axis_name="c", subcore_axis_name="s"),
    compiler_params=pltpu.CompilerParams(needs_layout_passes=False,
                                         disable_bounds_checks=True),
    out_type=jax.ShapeDtypeStruct((S, D, B), jnp.float32),
    scratch_types=[
        pltpu.VMEM((TROWS * TSTR,), jnp.float32),
        pltpu.VMEM((BW * S,), jnp.int32),
        pltpu.VMEM((SCH, D, BW), jnp.float32),
        pltpu.SemaphoreType.DMA,
    ],
)
def _sc_gather_t(tab_hbm, idx_hbm, out_hbm, tab_v, idx_v, out_v, sem):
    _gather_body(tab_hbm, idx_hbm, out_hbm, tab_v, idx_v, out_v, sem)


def kernel(atomic_numbers, embedding_weight, electron_config, config_w, config_b):
    table = _build_table(embedding_weight, electron_config, config_w, config_b)
    out_t = _sc_gather_t(table.reshape(TROWS * TSTR), atomic_numbers.reshape(N))
    return jnp.transpose(out_t, (2, 0, 1))


# speedup vs baseline: 5.2125x; 1.7714x over previous
import functools

import jax
import jax.numpy as jnp
from jax import lax
from jax.experimental import pallas as pl
from jax.experimental.pallas import tpu as pltpu
from jax.experimental.pallas import tpu_sc as plsc

MAXZ1 = 102
NCFG = 20
D = 64
B = 4096
S = 200
N = B * S
TROWS = 208
TSTR = 65
EMB_OFF = MAXZ1 * TSTR
NC = 2
NS = 16
NW = NC * NS
BW = B // NW
SCH = 4
NCHUNK = S // SCH


def _build_table(embedding_weight, electron_config, config_w, config_b):
    emb_p = (jnp.zeros((TROWS, D), jnp.float32)
             .at[:MAXZ1].set(embedding_weight)
             .at[MAXZ1:2 * MAXZ1].set(embedding_weight))
    ec_p = jnp.zeros((TROWS, 128), jnp.float32).at[:MAXZ1, :NCFG].set(electron_config)
    wt_p = jnp.zeros((128, D), jnp.float32).at[:NCFG].set(config_w.T)
    b_p = jnp.broadcast_to(config_b[None, :], (8, D))
    bmask = (lax.broadcasted_iota(jnp.int32, (TROWS, 1), 0) < MAXZ1).astype(jnp.float32)
    def body(emb_ref, ec_ref, wt_ref, b_ref, m_ref, out_ref):
        elec = jnp.dot(ec_ref[...], wt_ref[...], preferred_element_type=jnp.float32)
        out_ref[:, 0:D] = emb_ref[...] + elec + b_ref[0:1, :] * m_ref[...]
        out_ref[:, D:TSTR] = jnp.zeros((TROWS, TSTR - D), jnp.float32)
    return pl.pallas_call(
        body,
        out_shape=jax.ShapeDtypeStruct((TROWS, TSTR), jnp.float32),
    )(emb_p, ec_p, wt_p, b_p, bmask)


def _gather_body(tab_hbm, idx_hbm, out_hbm, tab_v, idx_v, out_v0, out_v1,
                 sem0, sem1):
    wid = lax.axis_index("s") * NC + lax.axis_index("c")
    b0 = wid * BW
    pltpu.sync_copy(tab_hbm, tab_v)
    pltpu.sync_copy(idx_hbm.at[pl.ds(b0 * S, BW * S)], idx_v)
    lanes = lax.broadcasted_iota(jnp.int32, (16,), 0)

    out_vs = (out_v0, out_v1)
    sems = (sem0, sem1)

    def fill(c, out_v):
        for s_loc in range(SCH):
            s = c * SCH + s_loc
            off = jnp.where(s == 0, EMB_OFF, 0)
            @plsc.parallel_loop(0, BW // 16, unroll=4)
            def bgroup(bg):
                bvec = bg * 16 + lanes
                z16 = plsc.load_gather(idx_v, [bvec * S + s])
                base = z16 * TSTR + off
                for d in range(D):
                    vals = plsc.load_gather(tab_v, [base + d])
                    out_v[s_loc, d, pl.ds(bg * 16, 16)] = vals

    def chunk2(k, carry):
        for par in range(2):
            c = 2 * k + par
            @pl.when(k > 0)
            def _drain():
                pltpu.make_async_copy(
                    out_vs[par],
                    out_hbm.at[pl.ds(0, SCH), :, pl.ds(b0, BW)],
                    sems[par]).wait()
            fill(c, out_vs[par])
            pltpu.async_copy(
                out_vs[par],
                out_hbm.at[pl.ds(c * SCH, SCH), :, pl.ds(b0, BW)],
                sems[par])
        return carry

    lax.fori_loop(0, NCHUNK // 2, chunk2, 0)
    for par in range(2):
        pltpu.make_async_copy(
            out_vs[par],
            out_hbm.at[pl.ds(0, SCH), :, pl.ds(b0, BW)],
            sems[par]).wait()


@functools.partial(
    pl.kernel,
    mesh=plsc.VectorSubcoreMesh(core_axis_name="c", subcore_axis_name="s"),
    compiler_params=pltpu.CompilerParams(needs_layout_passes=False,
                                         disable_bounds_checks=True),
    out_type=jax.ShapeDtypeStruct((S, D, B), jnp.float32),
    scratch_types=[
        pltpu.VMEM((TROWS * TSTR,), jnp.float32),
        pltpu.VMEM((BW * S,), jnp.int32),
        pltpu.VMEM((SCH, D, BW), jnp.float32),
        pltpu.VMEM((SCH, D, BW), jnp.float32),
        pltpu.SemaphoreType.DMA,
        pltpu.SemaphoreType.DMA,
    ],
)
def _sc_gather_t(tab_hbm, idx_hbm, out_hbm, tab_v, idx_v, out_v0, out_v1,
                 sem0, sem1):
    _gather_body(tab_hbm, idx_hbm, out_hbm, tab_v, idx_v, out_v0, out_v1,
                 sem0, sem1)


def kernel(atomic_numbers, embedding_weight, electron_config, config_w, config_b):
    table = _build_table(embedding_weight, electron_config, config_w, config_b)
    out_t = _sc_gather_t(table.reshape(TROWS * TSTR), atomic_numbers.reshape(N))
    return jnp.transpose(out_t, (2, 0, 1))
